# baseline (speedup 1.0000x reference)
import jax
import jax.numpy as jnp
from jax.experimental import pallas as pl
from jax.experimental.pallas import tpu as pltpu

D_MODEL = 128
BATCH = 32
TILE = 10000
HALF = TILE // 2


def _flash_update(q, k, v, m_ref, l_ref, acc_ref):
    s = jax.lax.dot_general(
        q, k, (((1,), (1,)), ((), ())), preferred_element_type=jnp.float32
    ) * (1.0 / (D_MODEL ** 0.5))
    m_prev = m_ref[...][:, 0:1]
    l_prev = l_ref[...][:, 0:1]
    m_cur = jnp.max(s, axis=1, keepdims=True)
    m_new = jnp.maximum(m_prev, m_cur)
    alpha = jnp.exp(m_prev - m_new)
    p = jnp.exp(s - m_new)
    l_new = alpha * l_prev + jnp.sum(p, axis=1, keepdims=True)
    pv = jax.lax.dot_general(
        p, v, (((1,), (0,)), ((), ())), preferred_element_type=jnp.float32
    )
    acc_ref[...] = acc_ref[...] * alpha + pv
    m_ref[...] = jnp.broadcast_to(m_new, m_ref.shape)
    l_ref[...] = jnp.broadcast_to(l_new, l_ref.shape)


def _attn_read_kernel(q_ref, ka_ref, kb_ref, va_ref, vb_ref, o_ref,
                      m_ref, l_ref, acc_ref):
    i = pl.program_id(0)
    n = pl.num_programs(0)

    @pl.when(i == 0)
    def _init():
        m_ref[...] = jnp.full(m_ref.shape, -jnp.inf, dtype=jnp.float32)
        l_ref[...] = jnp.zeros(l_ref.shape, dtype=jnp.float32)
        acc_ref[...] = jnp.zeros(acc_ref.shape, dtype=jnp.float32)

    q = q_ref[...]
    _flash_update(q, ka_ref[0], va_ref[0], m_ref, l_ref, acc_ref)
    _flash_update(q, kb_ref[0], vb_ref[0], m_ref, l_ref, acc_ref)

    @pl.when(i == n - 1)
    def _finish():
        o_ref[...] = acc_ref[...] / l_ref[...]


def kernel(hidden_state, keys, values):
    max_size = keys.shape[0]
    n_tiles = max_size // TILE
    k3 = keys.reshape(n_tiles, TILE, D_MODEL)
    v3 = values.reshape(n_tiles, TILE, D_MODEL)
    half_spec_a = pl.BlockSpec((1, HALF, D_MODEL), lambda i: (i, 0, 0))
    half_spec_b = pl.BlockSpec((1, HALF, D_MODEL), lambda i: (i, 1, 0))
    return pl.pallas_call(
        _attn_read_kernel,
        grid=(n_tiles,),
        in_specs=[
            pl.BlockSpec((BATCH, D_MODEL), lambda i: (0, 0)),
            half_spec_a,
            half_spec_b,
            half_spec_a,
            half_spec_b,
        ],
        out_specs=pl.BlockSpec((BATCH, D_MODEL), lambda i: (0, 0)),
        out_shape=jax.ShapeDtypeStruct((BATCH, D_MODEL), jnp.float32),
        scratch_shapes=[
            pltpu.VMEM((BATCH, D_MODEL), jnp.float32),
            pltpu.VMEM((BATCH, D_MODEL), jnp.float32),
            pltpu.VMEM((BATCH, D_MODEL), jnp.float32),
        ],
    )(hidden_state, k3, k3, v3, v3)


# speedup vs baseline: 1.0386x; 1.0386x over previous
import jax
import jax.numpy as jnp
from jax.experimental import pallas as pl
from jax.experimental.pallas import tpu as pltpu

D_MODEL = 128
BATCH = 32
BUFROWS = 4000
NBUF = 4
CHUNKS = [1000, 1000, 2000] + [4000] * 24
assert sum(CHUNKS) == 100000
_OFFS = [sum(CHUNKS[:i]) for i in range(len(CHUNKS))]


def _attn_read_kernel(q_ref, k_hbm, v_hbm, o_ref,
                      kbuf, vbuf, m_ref, l_ref, acc_ref, ksem, vsem):
    nchunks = len(CHUNKS)

    def k_copy(c):
        b = c % NBUF
        n = CHUNKS[c]
        return pltpu.make_async_copy(
            k_hbm.at[pl.ds(_OFFS[c], n)], kbuf.at[b, pl.ds(0, n)], ksem.at[b])

    def v_copy(c):
        b = c % NBUF
        n = CHUNKS[c]
        return pltpu.make_async_copy(
            v_hbm.at[pl.ds(_OFFS[c], n)], vbuf.at[b, pl.ds(0, n)], vsem.at[b])

    for c in range(NBUF):
        k_copy(c).start()
        v_copy(c).start()

    m_ref[...] = jnp.full(m_ref.shape, -jnp.inf, dtype=jnp.float32)
    l_ref[...] = jnp.zeros(l_ref.shape, dtype=jnp.float32)
    acc_ref[...] = jnp.zeros(acc_ref.shape, dtype=jnp.float32)

    q = q_ref[...]
    for c in range(nchunks):
        b = c % NBUF
        n = CHUNKS[c]
        k_copy(c).wait()
        v_copy(c).wait()
        k = kbuf[b, 0:n]
        v = vbuf[b, 0:n]
        s = jax.lax.dot_general(
            q, k, (((1,), (1,)), ((), ())), preferred_element_type=jnp.float32
        ) * (1.0 / (D_MODEL ** 0.5))
        m_prev = m_ref[...][:, 0:1]
        l_prev = l_ref[...][:, 0:1]
        m_cur = jnp.max(s, axis=1, keepdims=True)
        m_new = jnp.maximum(m_prev, m_cur)
        alpha = jnp.exp(m_prev - m_new)
        p = jnp.exp(s - m_new)
        l_new = alpha * l_prev + jnp.sum(p, axis=1, keepdims=True)
        pv = jax.lax.dot_general(
            p, v, (((1,), (0,)), ((), ())), preferred_element_type=jnp.float32
        )
        acc_ref[...] = acc_ref[...] * alpha + pv
        m_ref[...] = jnp.broadcast_to(m_new, m_ref.shape)
        l_ref[...] = jnp.broadcast_to(l_new, l_ref.shape)
        if c + NBUF < nchunks:
            k_copy(c + NBUF).start()
            v_copy(c + NBUF).start()

    o_ref[...] = acc_ref[...] / l_ref[...]


def kernel(hidden_state, keys, values):
    return pl.pallas_call(
        _attn_read_kernel,
        grid=(1,),
        in_specs=[
            pl.BlockSpec((BATCH, D_MODEL), lambda i: (0, 0)),
            pl.BlockSpec(memory_space=pl.ANY),
            pl.BlockSpec(memory_space=pl.ANY),
        ],
        out_specs=pl.BlockSpec((BATCH, D_MODEL), lambda i: (0, 0)),
        out_shape=jax.ShapeDtypeStruct((BATCH, D_MODEL), jnp.float32),
        scratch_shapes=[
            pltpu.VMEM((NBUF, BUFROWS, D_MODEL), jnp.float32),
            pltpu.VMEM((NBUF, BUFROWS, D_MODEL), jnp.float32),
            pltpu.VMEM((BATCH, D_MODEL), jnp.float32),
            pltpu.VMEM((BATCH, D_MODEL), jnp.float32),
            pltpu.VMEM((BATCH, D_MODEL), jnp.float32),
            pltpu.SemaphoreType.DMA((NBUF,)),
            pltpu.SemaphoreType.DMA((NBUF,)),
        ],
    )(hidden_state, keys, values)


# speedup vs baseline: 1.0486x; 1.0096x over previous
import jax
import jax.numpy as jnp
from jax.experimental import pallas as pl
from jax.experimental.pallas import tpu as pltpu

D_MODEL = 128
BATCH = 32
BUFROWS = 8000
NBUF = 4
CHUNKS = [1000, 1000, 2000, 4000] + [8000] * 11 + [4000]
assert sum(CHUNKS) == 100000
_OFFS = [sum(CHUNKS[:i]) for i in range(len(CHUNKS))]


def _attn_read_kernel(q_ref, k_hbm, v_hbm, o_ref, kbuf, vbuf, ksem, vsem):
    nchunks = len(CHUNKS)

    def k_copy(c):
        b = c % NBUF
        n = CHUNKS[c]
        return pltpu.make_async_copy(
            k_hbm.at[pl.ds(_OFFS[c], n)], kbuf.at[b, pl.ds(0, n)], ksem.at[b])

    def v_copy(c):
        b = c % NBUF
        n = CHUNKS[c]
        return pltpu.make_async_copy(
            v_hbm.at[pl.ds(_OFFS[c], n)], vbuf.at[b, pl.ds(0, n)], vsem.at[b])

    for c in range(NBUF):
        k_copy(c).start()
        v_copy(c).start()

    q = q_ref[...]
    m = jnp.full((BATCH, 1), -jnp.inf, dtype=jnp.float32)
    l = jnp.zeros((BATCH, 1), dtype=jnp.float32)
    acc = jnp.zeros((BATCH, D_MODEL), dtype=jnp.float32)

    for c in range(nchunks):
        b = c % NBUF
        n = CHUNKS[c]
        k_copy(c).wait()
        s = jax.lax.dot_general(
            q, kbuf[b, 0:n], (((1,), (1,)), ((), ())),
            preferred_element_type=jnp.float32,
        ) * (1.0 / (D_MODEL ** 0.5))
        m_new = jnp.maximum(m, jnp.max(s, axis=1, keepdims=True))
        alpha = jnp.exp(m - m_new)
        p = jnp.exp(s - m_new)
        l = alpha * l + jnp.sum(p, axis=1, keepdims=True)
        m = m_new
        v_copy(c).wait()
        pv = jax.lax.dot_general(
            p, vbuf[b, 0:n], (((1,), (0,)), ((), ())),
            preferred_element_type=jnp.float32,
        )
        acc = acc * alpha + pv
        if c + NBUF < nchunks:
            k_copy(c + NBUF).start()
            v_copy(c + NBUF).start()

    o_ref[...] = acc / l


def kernel(hidden_state, keys, values):
    return pl.pallas_call(
        _attn_read_kernel,
        grid=(1,),
        in_specs=[
            pl.BlockSpec((BATCH, D_MODEL), lambda i: (0, 0)),
            pl.BlockSpec(memory_space=pl.ANY),
            pl.BlockSpec(memory_space=pl.ANY),
        ],
        out_specs=pl.BlockSpec((BATCH, D_MODEL), lambda i: (0, 0)),
        out_shape=jax.ShapeDtypeStruct((BATCH, D_MODEL), jnp.float32),
        scratch_shapes=[
            pltpu.VMEM((NBUF, BUFROWS, D_MODEL), jnp.float32),
            pltpu.VMEM((NBUF, BUFROWS, D_MODEL), jnp.float32),
            pltpu.SemaphoreType.DMA((NBUF,)),
            pltpu.SemaphoreType.DMA((NBUF,)),
        ],
    )(hidden_state, keys, values)


# speedup vs baseline: 1.0511x; 1.0024x over previous
import jax
import jax.numpy as jnp
from jax.experimental import pallas as pl
from jax.experimental.pallas import tpu as pltpu

D_MODEL = 128
BATCH = 32
BUFROWS = 10000
NBUF = 4
CHUNKS = [1000, 1000, 2000, 4000, 8000] + [10000] * 8 + [4000]
assert sum(CHUNKS) == 100000
_OFFS = [sum(CHUNKS[:i]) for i in range(len(CHUNKS))]


def _attn_read_kernel(q_ref, k_hbm, v_hbm, o_ref, kbuf, vbuf, ksem, vsem):
    nchunks = len(CHUNKS)

    def k_copy(c):
        b = c % NBUF
        n = CHUNKS[c]
        return pltpu.make_async_copy(
            k_hbm.at[pl.ds(_OFFS[c], n)], kbuf.at[b, pl.ds(0, n)], ksem.at[b])

    def v_copy(c):
        b = c % NBUF
        n = CHUNKS[c]
        return pltpu.make_async_copy(
            v_hbm.at[pl.ds(_OFFS[c], n)], vbuf.at[b, pl.ds(0, n)], vsem.at[b])

    for c in range(NBUF):
        k_copy(c).start()
        v_copy(c).start()

    q = q_ref[...]
    m = jnp.full((BATCH, 1), -jnp.inf, dtype=jnp.float32)
    l = jnp.zeros((BATCH, 1), dtype=jnp.float32)
    acc = jnp.zeros((BATCH, D_MODEL), dtype=jnp.float32)

    for c in range(nchunks):
        b = c % NBUF
        n = CHUNKS[c]
        k_copy(c).wait()
        s = jax.lax.dot_general(
            q, kbuf[b, 0:n], (((1,), (1,)), ((), ())),
            preferred_element_type=jnp.float32,
        ) * (1.0 / (D_MODEL ** 0.5))
        m_new = jnp.maximum(m, jnp.max(s, axis=1, keepdims=True))
        alpha = jnp.exp(m - m_new)
        p = jnp.exp(s - m_new)
        l = alpha * l + jnp.sum(p, axis=1, keepdims=True)
        m = m_new
        v_copy(c).wait()
        pv = jax.lax.dot_general(
            p, vbuf[b, 0:n], (((1,), (0,)), ((), ())),
            preferred_element_type=jnp.float32,
        )
        acc = acc * alpha + pv
        if c + NBUF < nchunks:
            k_copy(c + NBUF).start()
            v_copy(c + NBUF).start()

    o_ref[...] = acc / l


def kernel(hidden_state, keys, values):
    return pl.pallas_call(
        _attn_read_kernel,
        grid=(1,),
        in_specs=[
            pl.BlockSpec((BATCH, D_MODEL), lambda i: (0, 0)),
            pl.BlockSpec(memory_space=pl.ANY),
            pl.BlockSpec(memory_space=pl.ANY),
        ],
        out_specs=pl.BlockSpec((BATCH, D_MODEL), lambda i: (0, 0)),
        out_shape=jax.ShapeDtypeStruct((BATCH, D_MODEL), jnp.float32),
        scratch_shapes=[
            pltpu.VMEM((NBUF, BUFROWS, D_MODEL), jnp.float32),
            pltpu.VMEM((NBUF, BUFROWS, D_MODEL), jnp.float32),
            pltpu.SemaphoreType.DMA((NBUF,)),
            pltpu.SemaphoreType.DMA((NBUF,)),
        ],
    )(hidden_state, keys, values)
